# baseline (speedup 1.0000x reference)
import jax
import jax.numpy as jnp
from jax.experimental import pallas as pl
from jax.experimental.pallas import tpu as pltpu


_N_CHUNKS = 16


def _copy_body(table_ref, o_ref, sems):
    n = o_ref.shape[0]
    rows = n // _N_CHUNKS
    for i in range(_N_CHUNKS):
        pltpu.make_async_copy(
            table_ref.at[pl.ds(i * rows, rows)],
            o_ref.at[pl.ds(i * rows, rows)],
            sems.at[i],
        ).start()
    for i in range(_N_CHUNKS):
        pltpu.make_async_copy(
            table_ref.at[pl.ds(i * rows, rows)],
            o_ref.at[pl.ds(i * rows, rows)],
            sems.at[i],
        ).wait()


def kernel(x, table):
    n = x.shape[1]
    return pl.pallas_call(
        _copy_body,
        out_shape=jax.ShapeDtypeStruct((n, table.shape[1]), table.dtype),
        in_specs=[pl.BlockSpec(memory_space=pl.ANY)],
        out_specs=pl.BlockSpec(memory_space=pl.ANY),
        scratch_shapes=[pltpu.SemaphoreType.DMA((_N_CHUNKS,))],
    )(table)


# speedup vs baseline: 41.5615x; 41.5615x over previous
import jax
import jax.numpy as jnp
from jax.experimental import pallas as pl
from jax.experimental.pallas import tpu as pltpu


_BLOCK_ROWS = 512


def _copy_body(table_ref, o_ref):
    o_ref[...] = table_ref[...]


def kernel(x, table):
    n = x.shape[1]
    d = table.shape[1]
    grid = (n // _BLOCK_ROWS,)
    return pl.pallas_call(
        _copy_body,
        out_shape=jax.ShapeDtypeStruct((n, d), table.dtype),
        grid=grid,
        in_specs=[pl.BlockSpec((_BLOCK_ROWS, d), lambda i: (i, 0))],
        out_specs=pl.BlockSpec((_BLOCK_ROWS, d), lambda i: (i, 0)),
    )(table)


# speedup vs baseline: 44.9159x; 1.0807x over previous
import jax
import jax.numpy as jnp
from jax.experimental import pallas as pl
from jax.experimental.pallas import tpu as pltpu


_BLOCK_ROWS = 1024


def _copy_body(table_ref, o_ref):
    o_ref[...] = table_ref[...]


def kernel(x, table):
    n = x.shape[1]
    d = table.shape[1]
    grid = (n // _BLOCK_ROWS,)
    return pl.pallas_call(
        _copy_body,
        out_shape=jax.ShapeDtypeStruct((n, d), table.dtype),
        grid=grid,
        in_specs=[pl.BlockSpec((_BLOCK_ROWS, d), lambda i: (i, 0))],
        out_specs=pl.BlockSpec((_BLOCK_ROWS, d), lambda i: (i, 0)),
    )(table)


# speedup vs baseline: 47.7544x; 1.0632x over previous
import jax
import jax.numpy as jnp
from jax.experimental import pallas as pl
from jax.experimental.pallas import tpu as pltpu


_BLOCK_ROWS = 2048


def _copy_body(table_ref, o_ref):
    o_ref[...] = table_ref[...]


def kernel(x, table):
    n = x.shape[1]
    d = table.shape[1]
    grid = (n // _BLOCK_ROWS,)
    return pl.pallas_call(
        _copy_body,
        out_shape=jax.ShapeDtypeStruct((n, d), table.dtype),
        grid=grid,
        in_specs=[pl.BlockSpec((_BLOCK_ROWS, d), lambda i: (i, 0))],
        out_specs=pl.BlockSpec((_BLOCK_ROWS, d), lambda i: (i, 0)),
    )(table)


# speedup vs baseline: 49.0703x; 1.0276x over previous
import jax
import jax.numpy as jnp
from jax.experimental import pallas as pl
from jax.experimental.pallas import tpu as pltpu


_CHUNK = 512
_NBUF = 8


def _copy_body(t_ref, o_ref, buf, rsems, wsems):
    n = o_ref.shape[0]
    num = n // _CHUNK

    def rd(i, s):
        return pltpu.make_async_copy(
            t_ref.at[pl.ds(i * _CHUNK, _CHUNK)], buf.at[s], rsems.at[s]
        )

    def wr(i, s):
        return pltpu.make_async_copy(
            buf.at[s], o_ref.at[pl.ds(i * _CHUNK, _CHUNK)], wsems.at[s]
        )

    depth = min(_NBUF, num)
    for s in range(depth):
        rd(s, s).start()
    for i in range(num):
        s = i % _NBUF
        rd(i, s).wait()
        wr(i, s).start()
        nxt = i + _NBUF
        if nxt < num:
            wr(i, s).wait()
            rd(nxt, s).start()
    for i in range(max(num - _NBUF, 0), num):
        wr(i, i % _NBUF).wait()


def kernel(x, table):
    n = x.shape[1]
    d = table.shape[1]
    return pl.pallas_call(
        _copy_body,
        out_shape=jax.ShapeDtypeStruct((n, d), table.dtype),
        in_specs=[pl.BlockSpec(memory_space=pl.ANY)],
        out_specs=pl.BlockSpec(memory_space=pl.ANY),
        scratch_shapes=[
            pltpu.VMEM((_NBUF, _CHUNK, 1024), jnp.float32),
            pltpu.SemaphoreType.DMA((_NBUF,)),
            pltpu.SemaphoreType.DMA((_NBUF,)),
        ],
    )(table)
